# Initial kernel scaffold; baseline (speedup 1.0000x reference)
import jax
import jax.numpy as jnp
from jax.experimental import pallas as pl


def kernel(x, centroids, values):
    raise NotImplementedError("write your pallas kernel here")


# speedup vs baseline: 20.3112x; 20.3112x over previous
import jax
import jax.numpy as jnp
from jax.experimental import pallas as pl
from jax.experimental.pallas import tpu as pltpu

_B, _C, _H, _W = 8, 96, 56, 56
_N, _VD, _K = 1024, 128, 8
_WP = 58
_QV = _H * _WP
_RQ = 8 * _WP
_NBLK = _H // 8
_PAD = 3368


def _nva_block(xf_ref, w_ref, v_ref, o_ref):
    i = pl.program_id(1)
    q0 = i * _RQ

    d = None
    for o in range(9):
        kh, kw = o // 3, o % 3
        off = kh * _WP + kw
        xs = xf_ref[0, pl.ds(q0 + off, _RQ), :]
        t = jax.lax.dot_general(
            xs, w_ref[o],
            (((1,), (0,)), ((), ())),
            preferred_element_type=jnp.float32,
            precision=jax.lax.Precision.DEFAULT,
        )
        d = t if d is None else d + t
    bias = 0.5 * jnp.sum(w_ref[...] ** 2, axis=(0, 1))
    d = d + bias[None, :]

    mask = jnp.zeros_like(d)
    k_rem = jnp.full((_RQ, 1), float(_K), dtype=jnp.float32)
    work = d
    for _ in range(_K):
        m = jnp.min(work, axis=1, keepdims=True)
        eq = (work == m)
        eqf = eq.astype(jnp.float32)
        c = jnp.sum(eqf, axis=1, keepdims=True)
        take = jnp.minimum(c, k_rem)
        mask = mask + eqf * (take / c)
        k_rem = k_rem - take
        work = jnp.where(eq, jnp.inf, work)
    mask = mask * (1.0 / _K)

    o_t = jax.lax.dot_general(
        mask, v_ref[...],
        (((1,), (0,)), ((), ())),
        preferred_element_type=jnp.float32,
        precision=jax.lax.Precision.HIGHEST,
    )
    o_ref[0] = o_t


def kernel(x, centroids, values):
    xt = jnp.transpose(x, (0, 2, 3, 1))
    xp = jnp.pad(xt, ((0, 0), (1, 1), (1, 1), (0, 0)))
    xf = xp.reshape(_B, _WP * _WP, _C)
    xf = jnp.pad(xf, ((0, 0), (0, _PAD - _WP * _WP), (0, 0)))
    wt = jnp.transpose(centroids, (2, 3, 1, 0)).reshape(9, _C, _N) * (-1.0)

    out = pl.pallas_call(
        _nva_block,
        grid=(_B, _NBLK),
        in_specs=[
            pl.BlockSpec((1, _PAD, _C), lambda b, i: (b, 0, 0)),
            pl.BlockSpec((9, _C, _N), lambda b, i: (0, 0, 0)),
            pl.BlockSpec((_N, _VD), lambda b, i: (0, 0)),
        ],
        out_specs=pl.BlockSpec((1, _RQ, _VD), lambda b, i: (b, i, 0)),
        out_shape=jax.ShapeDtypeStruct((_B, _QV, _VD), jnp.float32),
        compiler_params=pltpu.CompilerParams(
            dimension_semantics=("parallel", "parallel"),
        ),
    )(xf, wt, values)

    out = out.reshape(_B, _H, _WP, _VD)[:, :, :_W, :]
    return jnp.transpose(out, (0, 3, 1, 2))


# speedup vs baseline: 33.5415x; 1.6514x over previous
import jax
import jax.numpy as jnp
from jax.experimental import pallas as pl
from jax.experimental.pallas import tpu as pltpu

_B, _C, _H, _W = 8, 96, 56, 56
_N, _VD, _K = 1024, 128, 8
_WP = 58
_QV = _H * _WP
_RQ = 8 * _WP
_NBLK = _H // 8
_PAD = 3368


def _nva_block(xf_ref, w_ref, v_ref, o_ref):
    i = pl.program_id(1)
    q0 = i * _RQ

    xcat = jnp.concatenate(
        [xf_ref[0, pl.ds(q0 + kh * _WP + kw, _RQ), :]
         for kh in range(3) for kw in range(3)],
        axis=1,
    )
    d = jax.lax.dot_general(
        xcat, w_ref[...],
        (((1,), (0,)), ((), ())),
        preferred_element_type=jnp.float32,
        precision=jax.lax.Precision.DEFAULT,
    )
    bias = 0.5 * jnp.sum(w_ref[...] ** 2, axis=0)
    d = d + bias[None, :]

    mask = jnp.zeros_like(d)
    k_rem = jnp.full((_RQ, 1), float(_K), dtype=jnp.float32)
    work = d
    for _ in range(_K):
        m = jnp.min(work, axis=1, keepdims=True)
        eq = (work == m)
        eqf = eq.astype(jnp.float32)
        c = jnp.sum(eqf, axis=1, keepdims=True)
        take = jnp.minimum(c, k_rem)
        mask = mask + eqf * (take / c)
        k_rem = k_rem - take
        work = jnp.where(eq, jnp.inf, work)
    mask = mask * (1.0 / _K)

    o_t = jax.lax.dot_general(
        mask, v_ref[...],
        (((1,), (0,)), ((), ())),
        preferred_element_type=jnp.float32,
        precision=jax.lax.Precision.DEFAULT,
    )
    o_ref[0] = o_t


def kernel(x, centroids, values):
    xt = jnp.transpose(x, (0, 2, 3, 1))
    xp = jnp.pad(xt, ((0, 0), (1, 1), (1, 1), (0, 0)))
    xf = xp.reshape(_B, _WP * _WP, _C)
    xf = jnp.pad(xf, ((0, 0), (0, _PAD - _WP * _WP), (0, 0)))
    wt = jnp.transpose(centroids, (2, 3, 1, 0)).reshape(9 * _C, _N) * (-1.0)

    out = pl.pallas_call(
        _nva_block,
        grid=(_B, _NBLK),
        in_specs=[
            pl.BlockSpec((1, _PAD, _C), lambda b, i: (b, 0, 0)),
            pl.BlockSpec((9 * _C, _N), lambda b, i: (0, 0)),
            pl.BlockSpec((_N, _VD), lambda b, i: (0, 0)),
        ],
        out_specs=pl.BlockSpec((1, _RQ, _VD), lambda b, i: (b, i, 0)),
        out_shape=jax.ShapeDtypeStruct((_B, _QV, _VD), jnp.float32),
        compiler_params=pltpu.CompilerParams(
            dimension_semantics=("parallel", "parallel"),
        ),
    )(xf, wt, values)

    out = out.reshape(_B, _H, _WP, _VD)[:, :, :_W, :]
    return jnp.transpose(out, (0, 3, 1, 2))


# speedup vs baseline: 42.9665x; 1.2810x over previous
import jax
import jax.numpy as jnp
from jax.experimental import pallas as pl
from jax.experimental.pallas import tpu as pltpu

_B, _C, _H, _W = 8, 96, 56, 56
_N, _VD, _K = 1024, 128, 8
_WP = 58
_QV = _H * _WP
_RQ = 8 * _WP
_NBLK = _H // 8
_PAD = 3368


def _nva_block(xf_ref, w_ref, v_ref, o_ref):
    i = pl.program_id(1)
    q0 = i * _RQ

    xcat = jnp.concatenate(
        [xf_ref[0, pl.ds(q0 + kh * _WP + kw, _RQ), :]
         for kh in range(3) for kw in range(3)],
        axis=1,
    )
    d = jax.lax.dot_general(
        xcat, w_ref[...],
        (((1,), (0,)), ((), ())),
        preferred_element_type=jnp.float32,
        precision=jax.lax.Precision.DEFAULT,
    )
    bias = 0.5 * jnp.sum(w_ref[...] ** 2, axis=0)
    d = d + bias[None, :]

    m = jnp.min(d, axis=1, keepdims=True)
    for _ in range(_K - 1):
        m = jnp.min(jnp.where(d > m, d, jnp.inf), axis=1, keepdims=True)
    lt = (d < m).astype(jnp.float32)
    eq = (d == m).astype(jnp.float32)
    n_lt = jnp.sum(lt, axis=1, keepdims=True)
    n_eq = jnp.sum(eq, axis=1, keepdims=True)
    take = jnp.clip(float(_K) - n_lt, 0.0, None)
    mask = lt + eq * (take / n_eq)

    o_t = jax.lax.dot_general(
        mask, v_ref[...],
        (((1,), (0,)), ((), ())),
        preferred_element_type=jnp.float32,
        precision=jax.lax.Precision.DEFAULT,
    )
    o_ref[0] = o_t * (1.0 / _K)


def kernel(x, centroids, values):
    xt = jnp.transpose(x, (0, 2, 3, 1))
    xp = jnp.pad(xt, ((0, 0), (1, 1), (1, 1), (0, 0)))
    xf = xp.reshape(_B, _WP * _WP, _C)
    xf = jnp.pad(xf, ((0, 0), (0, _PAD - _WP * _WP), (0, 0)))
    wt = jnp.transpose(centroids, (2, 3, 1, 0)).reshape(9 * _C, _N) * (-1.0)

    out = pl.pallas_call(
        _nva_block,
        grid=(_B, _NBLK),
        in_specs=[
            pl.BlockSpec((1, _PAD, _C), lambda b, i: (b, 0, 0)),
            pl.BlockSpec((9 * _C, _N), lambda b, i: (0, 0)),
            pl.BlockSpec((_N, _VD), lambda b, i: (0, 0)),
        ],
        out_specs=pl.BlockSpec((1, _RQ, _VD), lambda b, i: (b, i, 0)),
        out_shape=jax.ShapeDtypeStruct((_B, _QV, _VD), jnp.float32),
        compiler_params=pltpu.CompilerParams(
            dimension_semantics=("parallel", "parallel"),
        ),
    )(xf, wt, values)

    out = out.reshape(_B, _H, _WP, _VD)[:, :, :_W, :]
    return jnp.transpose(out, (0, 3, 1, 2))


# speedup vs baseline: 46.6390x; 1.0855x over previous
import jax
import jax.numpy as jnp
from jax.experimental import pallas as pl
from jax.experimental.pallas import tpu as pltpu

_B, _C, _H, _W = 8, 96, 56, 56
_N, _VD, _K = 1024, 128, 8
_WP = 58
_QV = _H * _WP
_RQ = 8 * _WP
_NBLK = _H // 8
_PAD = 3368


def _nva_block(xf_ref, w_ref, b_ref, v_ref, o_ref):
    i = pl.program_id(1)
    q0 = i * _RQ

    xcat = jnp.concatenate(
        [xf_ref[0, pl.ds(q0 + kh * _WP + kw, _RQ), :].astype(jnp.bfloat16)
         for kh in range(3) for kw in range(3)],
        axis=1,
    )
    d = jax.lax.dot_general(
        xcat, w_ref[...],
        (((1,), (0,)), ((), ())),
        preferred_element_type=jnp.float32,
        precision=jax.lax.Precision.DEFAULT,
    )
    d = d + b_ref[...]

    m = jnp.min(d, axis=1, keepdims=True)
    for _ in range(_K - 1):
        m = jnp.min(jnp.where(d > m, d, jnp.inf), axis=1, keepdims=True)
    mask = (d <= m).astype(jnp.bfloat16)

    o_ref[0] = jax.lax.dot_general(
        mask, v_ref[...],
        (((1,), (0,)), ((), ())),
        preferred_element_type=jnp.float32,
        precision=jax.lax.Precision.DEFAULT,
    )


def kernel(x, centroids, values):
    xt = jnp.transpose(x, (0, 2, 3, 1))
    xp = jnp.pad(xt, ((0, 0), (1, 1), (1, 1), (0, 0)))
    xf = xp.reshape(_B, _WP * _WP, _C)
    xf = jnp.pad(xf, ((0, 0), (0, _PAD - _WP * _WP), (0, 0)))
    wt = jnp.transpose(-centroids, (2, 3, 1, 0)).reshape(9 * _C, _N)
    wt = wt.astype(jnp.bfloat16)
    bias = 0.5 * jnp.sum(centroids.reshape(_N, -1) ** 2, axis=1)[None, :]
    vs = (values * (1.0 / _K)).astype(jnp.bfloat16)

    out = pl.pallas_call(
        _nva_block,
        grid=(_B, _NBLK),
        in_specs=[
            pl.BlockSpec((1, _PAD, _C), lambda b, i: (b, 0, 0)),
            pl.BlockSpec((9 * _C, _N), lambda b, i: (0, 0)),
            pl.BlockSpec((1, _N), lambda b, i: (0, 0)),
            pl.BlockSpec((_N, _VD), lambda b, i: (0, 0)),
        ],
        out_specs=pl.BlockSpec((1, _RQ, _VD), lambda b, i: (b, i, 0)),
        out_shape=jax.ShapeDtypeStruct((_B, _QV, _VD), jnp.float32),
        compiler_params=pltpu.CompilerParams(
            dimension_semantics=("parallel", "parallel"),
        ),
    )(xf, wt, bias, vs)

    out = out.reshape(_B, _H, _WP, _VD)[:, :, :_W, :]
    return jnp.transpose(out, (0, 3, 1, 2))


# speedup vs baseline: 49.7556x; 1.0668x over previous
import jax
import jax.numpy as jnp
from jax.experimental import pallas as pl
from jax.experimental.pallas import tpu as pltpu

_B, _C, _H, _W = 8, 96, 56, 56
_N, _VD, _K = 1024, 128, 8
_WP = 58
_QV = _H * _WP
_RQ = 8 * _WP
_NBLK = _H // 8
_PAD = 3368


def _nva_block(xf_ref, w_ref, b_ref, v_ref, o_ref, d2_ref):
    s = pl.program_id(1)

    @pl.when(s < _NBLK)
    def _produce():
        q0 = s * _RQ
        xcat = jnp.concatenate(
            [xf_ref[0, pl.ds(q0 + kh * _WP + kw, _RQ), :].astype(jnp.bfloat16)
             for kh in range(3) for kw in range(3)],
            axis=1,
        )
        d = jax.lax.dot_general(
            xcat, w_ref[...],
            (((1,), (0,)), ((), ())),
            preferred_element_type=jnp.float32,
            precision=jax.lax.Precision.DEFAULT,
        )
        d2_ref[pl.ds(jax.lax.rem(s, 2), 1)] = (d + b_ref[...])[None]

    @pl.when(s > 0)
    def _consume():
        d = d2_ref[pl.ds(jax.lax.rem(s + 1, 2), 1)][0]
        m = jnp.min(d, axis=1, keepdims=True)
        for _ in range(_K - 1):
            m = jnp.min(jnp.where(d > m, d, jnp.inf), axis=1, keepdims=True)
        mask = (d <= m).astype(jnp.bfloat16)
        o_ref[0] = jax.lax.dot_general(
            mask, v_ref[...],
            (((1,), (0,)), ((), ())),
            preferred_element_type=jnp.float32,
            precision=jax.lax.Precision.DEFAULT,
        )


def kernel(x, centroids, values):
    xt = jnp.transpose(x, (0, 2, 3, 1))
    xp = jnp.pad(xt, ((0, 0), (1, 1), (1, 1), (0, 0)))
    xf = xp.reshape(_B, _WP * _WP, _C)
    xf = jnp.pad(xf, ((0, 0), (0, _PAD - _WP * _WP), (0, 0)))
    wt = jnp.transpose(-centroids, (2, 3, 1, 0)).reshape(9 * _C, _N)
    wt = wt.astype(jnp.bfloat16)
    bias = 0.5 * jnp.sum(centroids.reshape(_N, -1) ** 2, axis=1)[None, :]
    vs = (values * (1.0 / _K)).astype(jnp.bfloat16)

    out = pl.pallas_call(
        _nva_block,
        grid=(_B, _NBLK + 1),
        in_specs=[
            pl.BlockSpec((1, _PAD, _C), lambda b, s: (b, 0, 0)),
            pl.BlockSpec((9 * _C, _N), lambda b, s: (0, 0)),
            pl.BlockSpec((1, _N), lambda b, s: (0, 0)),
            pl.BlockSpec((_N, _VD), lambda b, s: (0, 0)),
        ],
        out_specs=pl.BlockSpec(
            (1, _RQ, _VD),
            lambda b, s: (b, jnp.maximum(s - 1, 0), 0),
        ),
        out_shape=jax.ShapeDtypeStruct((_B, _QV, _VD), jnp.float32),
        scratch_shapes=[pltpu.VMEM((2, _RQ, _N), jnp.float32)],
        compiler_params=pltpu.CompilerParams(
            dimension_semantics=("parallel", "arbitrary"),
        ),
    )(xf, wt, bias, vs)

    out = out.reshape(_B, _H, _WP, _VD)[:, :, :_W, :]
    return jnp.transpose(out, (0, 3, 1, 2))


# speedup vs baseline: 54.1740x; 1.0888x over previous
import jax
import jax.numpy as jnp
from jax.experimental import pallas as pl
from jax.experimental.pallas import tpu as pltpu

_B, _C, _H, _W = 8, 96, 56, 56
_N, _VD, _K = 1024, 128, 8
_WP = 58
_QV = _H * _WP
_RQ = 28 * _WP
_NBLK = _H // 28
_PAD = 3368


def _nva_block(xf_ref, w_ref, b_ref, v_ref, o_ref, d2_ref):
    s = pl.program_id(1)

    @pl.when(s < _NBLK)
    def _produce():
        q0 = s * _RQ
        xcat = jnp.concatenate(
            [xf_ref[0, pl.ds(q0 + kh * _WP + kw, _RQ), :].astype(jnp.bfloat16)
             for kh in range(3) for kw in range(3)],
            axis=1,
        )
        d = jax.lax.dot_general(
            xcat, w_ref[...],
            (((1,), (0,)), ((), ())),
            preferred_element_type=jnp.float32,
            precision=jax.lax.Precision.DEFAULT,
        )
        d2_ref[pl.ds(jax.lax.rem(s, 2), 1)] = (d + b_ref[...])[None]

    @pl.when(s > 0)
    def _consume():
        d = d2_ref[pl.ds(jax.lax.rem(s + 1, 2), 1)][0]
        m = jnp.min(d, axis=1, keepdims=True)
        for _ in range(_K - 1):
            m = jnp.min(jnp.where(d > m, d, jnp.inf), axis=1, keepdims=True)
        mask = (d <= m).astype(jnp.bfloat16)
        o_ref[0] = jax.lax.dot_general(
            mask, v_ref[...],
            (((1,), (0,)), ((), ())),
            preferred_element_type=jnp.float32,
            precision=jax.lax.Precision.DEFAULT,
        )


def kernel(x, centroids, values):
    xt = jnp.transpose(x, (0, 2, 3, 1))
    xp = jnp.pad(xt, ((0, 0), (1, 1), (1, 1), (0, 0)))
    xf = xp.reshape(_B, _WP * _WP, _C)
    xf = jnp.pad(xf, ((0, 0), (0, _PAD - _WP * _WP), (0, 0)))
    wt = jnp.transpose(-centroids, (2, 3, 1, 0)).reshape(9 * _C, _N)
    wt = wt.astype(jnp.bfloat16)
    bias = 0.5 * jnp.sum(centroids.reshape(_N, -1) ** 2, axis=1)[None, :]
    vs = (values * (1.0 / _K)).astype(jnp.bfloat16)

    out = pl.pallas_call(
        _nva_block,
        grid=(_B, _NBLK + 1),
        in_specs=[
            pl.BlockSpec((1, _PAD, _C), lambda b, s: (b, 0, 0)),
            pl.BlockSpec((9 * _C, _N), lambda b, s: (0, 0)),
            pl.BlockSpec((1, _N), lambda b, s: (0, 0)),
            pl.BlockSpec((_N, _VD), lambda b, s: (0, 0)),
        ],
        out_specs=pl.BlockSpec(
            (1, _RQ, _VD),
            lambda b, s: (b, jnp.maximum(s - 1, 0), 0),
        ),
        out_shape=jax.ShapeDtypeStruct((_B, _QV, _VD), jnp.float32),
        scratch_shapes=[pltpu.VMEM((2, _RQ, _N), jnp.float32)],
        compiler_params=pltpu.CompilerParams(
            dimension_semantics=("parallel", "arbitrary"),
        ),
    )(xf, wt, bias, vs)

    out = out.reshape(_B, _H, _WP, _VD)[:, :, :_W, :]
    return jnp.transpose(out, (0, 3, 1, 2))


# speedup vs baseline: 54.2078x; 1.0006x over previous
import jax
import jax.numpy as jnp
from jax.experimental import pallas as pl
from jax.experimental.pallas import tpu as pltpu

_B, _C, _H, _W = 8, 96, 56, 56
_N, _VD, _K = 1024, 128, 8
_WP = 58
_QV = _H * _WP
_RQ = 28 * _WP
_NBLK = _H // 28
_PAD = 3368


_NCHUNK = _B * _NBLK


def _nva_block(xf_ref, w_ref, b_ref, v_ref, o_ref, d2_ref):
    s = pl.program_id(0)

    @pl.when(s < _NCHUNK)
    def _produce():
        q0 = jax.lax.rem(s, _NBLK) * _RQ
        xcat = jnp.concatenate(
            [xf_ref[0, pl.ds(q0 + kh * _WP + kw, _RQ), :].astype(jnp.bfloat16)
             for kh in range(3) for kw in range(3)],
            axis=1,
        )
        d = jax.lax.dot_general(
            xcat, w_ref[...],
            (((1,), (0,)), ((), ())),
            preferred_element_type=jnp.float32,
            precision=jax.lax.Precision.DEFAULT,
        )
        d2_ref[pl.ds(jax.lax.rem(s, 2), 1)] = (d + b_ref[...])[None]

    @pl.when(s > 0)
    def _consume():
        d = d2_ref[pl.ds(jax.lax.rem(s + 1, 2), 1)][0]
        m = jnp.min(d, axis=1, keepdims=True)
        for _ in range(_K - 1):
            m = jnp.min(jnp.where(d > m, d, jnp.inf), axis=1, keepdims=True)
        mask = (d <= m).astype(jnp.bfloat16)
        o_ref[0] = jax.lax.dot_general(
            mask, v_ref[...],
            (((1,), (0,)), ((), ())),
            preferred_element_type=jnp.float32,
            precision=jax.lax.Precision.DEFAULT,
        )


def kernel(x, centroids, values):
    xt = jnp.transpose(x, (0, 2, 3, 1))
    xp = jnp.pad(xt, ((0, 0), (1, 1), (1, 1), (0, 0)))
    xf = xp.reshape(_B, _WP * _WP, _C)
    xf = jnp.pad(xf, ((0, 0), (0, _PAD - _WP * _WP), (0, 0)))
    wt = jnp.transpose(-centroids, (2, 3, 1, 0)).reshape(9 * _C, _N)
    wt = wt.astype(jnp.bfloat16)
    bias = 0.5 * jnp.sum(centroids.reshape(_N, -1) ** 2, axis=1)[None, :]
    vs = (values * (1.0 / _K)).astype(jnp.bfloat16)

    out = pl.pallas_call(
        _nva_block,
        grid=(_NCHUNK + 1,),
        in_specs=[
            pl.BlockSpec(
                (1, _PAD, _C),
                lambda s: (jnp.minimum(s // _NBLK, _B - 1), 0, 0),
            ),
            pl.BlockSpec((9 * _C, _N), lambda s: (0, 0)),
            pl.BlockSpec((1, _N), lambda s: (0, 0)),
            pl.BlockSpec((_N, _VD), lambda s: (0, 0)),
        ],
        out_specs=pl.BlockSpec(
            (1, _RQ, _VD),
            lambda s: (jnp.maximum(s - 1, 0) // _NBLK,
                       jnp.maximum(s - 1, 0) % _NBLK, 0),
        ),
        out_shape=jax.ShapeDtypeStruct((_B, _QV, _VD), jnp.float32),
        scratch_shapes=[pltpu.VMEM((2, _RQ, _N), jnp.float32)],
        compiler_params=pltpu.CompilerParams(
            dimension_semantics=("arbitrary",),
        ),
    )(xf, wt, bias, vs)

    out = out.reshape(_B, _H, _WP, _VD)[:, :, :_W, :]
    return jnp.transpose(out, (0, 3, 1, 2))


# speedup vs baseline: 55.7213x; 1.0279x over previous
import jax
import jax.numpy as jnp
from jax.experimental import pallas as pl
from jax.experimental.pallas import tpu as pltpu

_B, _C, _H, _W = 8, 96, 56, 56
_N, _VD, _K = 1024, 128, 8
_WP = 58
_QV = _H * _WP
_RQ = _H * _WP
_NBLK = 1
_PAD = 3368


_NCHUNK = _B * _NBLK


def _nva_block(xf_ref, w_ref, b_ref, v_ref, o_ref, d2_ref):
    s = pl.program_id(0)

    @pl.when(s < _NCHUNK)
    def _produce():
        q0 = jax.lax.rem(s, _NBLK) * _RQ
        xcat = jnp.concatenate(
            [xf_ref[0, pl.ds(q0 + kh * _WP + kw, _RQ), :].astype(jnp.bfloat16)
             for kh in range(3) for kw in range(3)],
            axis=1,
        )
        d = jax.lax.dot_general(
            xcat, w_ref[...],
            (((1,), (0,)), ((), ())),
            preferred_element_type=jnp.float32,
            precision=jax.lax.Precision.DEFAULT,
        )
        d2_ref[pl.ds(jax.lax.rem(s, 2), 1)] = (d + b_ref[...])[None]

    @pl.when(s > 0)
    def _consume():
        d = d2_ref[pl.ds(jax.lax.rem(s + 1, 2), 1)][0]
        m = jnp.min(d, axis=1, keepdims=True)
        for _ in range(_K - 1):
            m = jnp.min(jnp.where(d > m, d, jnp.inf), axis=1, keepdims=True)
        mask = (d <= m).astype(jnp.bfloat16)
        o_ref[0] = jax.lax.dot_general(
            v_ref[...], mask,
            (((0,), (1,)), ((), ())),
            preferred_element_type=jnp.float32,
            precision=jax.lax.Precision.DEFAULT,
        )


def kernel(x, centroids, values):
    xt = jnp.transpose(x, (0, 2, 3, 1))
    xp = jnp.pad(xt, ((0, 0), (1, 1), (1, 1), (0, 0)))
    xf = xp.reshape(_B, _WP * _WP, _C)
    xf = jnp.pad(xf, ((0, 0), (0, _PAD - _WP * _WP), (0, 0)))
    wt = jnp.transpose(-centroids, (2, 3, 1, 0)).reshape(9 * _C, _N)
    wt = wt.astype(jnp.bfloat16)
    bias = 0.5 * jnp.sum(centroids.reshape(_N, -1) ** 2, axis=1)[None, :]
    vs = (values * (1.0 / _K)).astype(jnp.bfloat16)

    out = pl.pallas_call(
        _nva_block,
        grid=(_NCHUNK + 1,),
        in_specs=[
            pl.BlockSpec(
                (1, _PAD, _C),
                lambda s: (jnp.minimum(s // _NBLK, _B - 1), 0, 0),
            ),
            pl.BlockSpec((9 * _C, _N), lambda s: (0, 0)),
            pl.BlockSpec((1, _N), lambda s: (0, 0)),
            pl.BlockSpec((_N, _VD), lambda s: (0, 0)),
        ],
        out_specs=pl.BlockSpec(
            (1, _VD, _RQ),
            lambda s: (jnp.maximum(s - 1, 0), 0, 0),
        ),
        out_shape=jax.ShapeDtypeStruct((_B, _VD, _QV), jnp.float32),
        scratch_shapes=[pltpu.VMEM((2, _RQ, _N), jnp.float32)],
        compiler_params=pltpu.CompilerParams(
            dimension_semantics=("arbitrary",),
        ),
    )(xf, wt, bias, vs)

    return out.reshape(_B, _VD, _H, _WP)[:, :, :, :_W]


# speedup vs baseline: 79.9301x; 1.4345x over previous
import jax
import jax.numpy as jnp
from jax.experimental import pallas as pl
from jax.experimental.pallas import tpu as pltpu

_B, _C, _H, _W = 8, 96, 56, 56
_N, _VD, _K = 1024, 128, 8
_WP = 58
_RQ = _H * _WP
_PAD = 3368


def _nva_block(xt_ref, w_ref, b_ref, v_ref, o_ref, xp_ref, d2_ref):
    s = pl.program_id(0)

    @pl.when(s == 0)
    def _zero_pad_buffer():
        xp_ref[...] = jnp.zeros((_PAD, _C), jnp.float32)

    @pl.when(s < _B)
    def _produce():
        for h in range(_H):
            xp_ref[pl.ds((h + 1) * _WP + 1, _W), :] = xt_ref[0, h]
        xcat = jnp.concatenate(
            [xp_ref[pl.ds(kh * _WP + kw, _RQ), :].astype(jnp.bfloat16)
             for kh in range(3) for kw in range(3)],
            axis=1,
        )
        d = jax.lax.dot_general(
            xcat, w_ref[...],
            (((1,), (0,)), ((), ())),
            preferred_element_type=jnp.float32,
            precision=jax.lax.Precision.DEFAULT,
        )
        d2_ref[pl.ds(jax.lax.rem(s, 2), 1)] = (d + b_ref[...])[None]

    @pl.when(s > 0)
    def _consume():
        d = d2_ref[pl.ds(jax.lax.rem(s + 1, 2), 1)][0]
        m = jnp.min(d, axis=1, keepdims=True)
        for _ in range(_K - 1):
            m = jnp.min(jnp.where(d > m, d, jnp.inf), axis=1, keepdims=True)
        mask = (d <= m).astype(jnp.bfloat16)
        o_t = jax.lax.dot_general(
            v_ref[...], mask,
            (((0,), (1,)), ((), ())),
            preferred_element_type=jnp.float32,
            precision=jax.lax.Precision.DEFAULT,
        )
        for h in range(_H):
            o_ref[0, :, pl.ds(h * _W, _W)] = o_t[:, h * _WP:h * _WP + _W]


def kernel(x, centroids, values):
    xt = jnp.transpose(x, (0, 2, 3, 1))
    wt = jnp.transpose(-centroids, (2, 3, 1, 0)).reshape(9 * _C, _N)
    wt = wt.astype(jnp.bfloat16)
    bias = 0.5 * jnp.sum(centroids.reshape(_N, -1) ** 2, axis=1)[None, :]
    vs = (values * (1.0 / _K)).astype(jnp.bfloat16)

    out = pl.pallas_call(
        _nva_block,
        grid=(_B + 1,),
        in_specs=[
            pl.BlockSpec((1, _H, _W, _C), lambda s: (jnp.minimum(s, _B - 1), 0, 0, 0)),
            pl.BlockSpec((9 * _C, _N), lambda s: (0, 0)),
            pl.BlockSpec((1, _N), lambda s: (0, 0)),
            pl.BlockSpec((_N, _VD), lambda s: (0, 0)),
        ],
        out_specs=pl.BlockSpec(
            (1, _VD, _H * _W),
            lambda s: (jnp.maximum(s - 1, 0), 0, 0),
        ),
        out_shape=jax.ShapeDtypeStruct((_B, _VD, _H * _W), jnp.float32),
        scratch_shapes=[
            pltpu.VMEM((_PAD, _C), jnp.float32),
            pltpu.VMEM((2, _RQ, _N), jnp.float32),
        ],
        compiler_params=pltpu.CompilerParams(
            dimension_semantics=("arbitrary",),
        ),
    )(xt, wt, bias, vs)

    return out.reshape(_B, _VD, _H, _W)
